# baseline (speedup 1.0000x reference)
import jax
import jax.numpy as jnp
from jax import lax
from jax.experimental import pallas as pl
from jax.experimental.pallas import tpu as pltpu
from jax.experimental.pallas import tpu_sc as plsc

BATCH = 16384
N_FIELDS = 26
VOCAB = 100000
EMBED_DIM = 32

NUM_PAIRS = N_FIELDS * EMBED_DIM
NUM_WORKERS = 32
PAIRS_PER_W = NUM_PAIRS // NUM_WORKERS
OCHUNK = 4096
N_OCHUNKS = BATCH // OCHUNK
LANES = 16
UNROLL = 8
ELEMS = LANES * UNROLL


def _emb_body(idx_hbm, tab_hbm, out_hbm, row_v, idx_v, out_v, osem):
    cid = lax.axis_index("c")
    sid = lax.axis_index("s")
    wid = sid * 2 + cid
    p0 = wid * PAIRS_PER_W

    def pair_body(j, carry):
        p = p0 + j
        f = p // EMBED_DIM
        d = lax.rem(p, EMBED_DIM)

        @pl.when(jnp.logical_or(j == 0, d == 0))
        def _():
            pltpu.sync_copy(idx_hbm.at[f], idx_v)

        pltpu.sync_copy(tab_hbm.at[f, d], row_v)

        handles = [None, None]
        for c in range(N_OCHUNKS):
            buf = c & 1
            if handles[buf] is not None:
                handles[buf].wait()

            def bgroup(i, carry2, c=c, buf=buf):
                base = i * ELEMS
                for u in range(UNROLL):
                    off = base + u * LANES
                    iv = idx_v[pl.ds(c * OCHUNK + off, LANES)]
                    out_v[buf, pl.ds(off, LANES)] = plsc.load_gather(row_v, [iv])
                return carry2

            lax.fori_loop(0, OCHUNK // ELEMS, bgroup, 0)
            handles[buf] = pltpu.async_copy(
                out_v.at[buf],
                out_hbm.at[f, d, pl.ds(c * OCHUNK, OCHUNK)],
                osem,
            )
        handles[0].wait()
        handles[1].wait()
        return carry

    lax.fori_loop(0, PAIRS_PER_W, pair_body, 0)


_emb_call = pl.kernel(
    _emb_body,
    mesh=plsc.VectorSubcoreMesh(core_axis_name="c", subcore_axis_name="s"),
    out_type=jax.ShapeDtypeStruct((N_FIELDS, EMBED_DIM, BATCH), jnp.float32),
    scratch_types=[
        pltpu.VMEM((VOCAB,), jnp.float32),
        pltpu.VMEM((BATCH,), jnp.int32),
        pltpu.VMEM((2, OCHUNK), jnp.float32),
        pltpu.SemaphoreType.DMA,
    ],
    compiler_params=pltpu.CompilerParams(needs_layout_passes=False),
)


@jax.jit
def kernel(indices, tables):
    idx_t = indices.astype(jnp.int32).T
    tab_t = jnp.transpose(tables, (0, 2, 1))
    out_t = _emb_call(idx_t, tab_t)
    return jnp.transpose(out_t, (2, 0, 1))
